# baseline (speedup 1.0000x reference)
import jax
import jax.numpy as jnp
from jax.experimental import pallas as pl
from jax.experimental.pallas import tpu as pltpu

_CHUNK = 4000
_NBUF = 8


def _colmax_body(x_hbm, o_ref, acc, buf, sem):
    n = x_hbm.shape[0] // _CHUNK

    def copy(i):
        return pltpu.make_async_copy(
            x_hbm.at[pl.ds(i * _CHUNK, _CHUNK), :],
            buf.at[i % _NBUF],
            sem.at[i % _NBUF],
        )

    for i in range(min(_NBUF, n)):
        copy(i).start()
    acc[...] = jnp.full(acc.shape, -jnp.inf, acc.dtype)
    for i in range(n):
        copy(i).wait()
        chunk = buf[i % _NBUF].reshape(_CHUNK // 32, 32, 128)
        acc[...] = jnp.maximum(acc[...], jnp.max(chunk, axis=0))
        j = i + _NBUF
        if j < n:
            copy(j).start()
    a = jnp.max(acc[...].reshape(4, 8, 128), axis=0)
    o_ref[...] = jnp.max(a, axis=0, keepdims=True)


def kernel(X):
    rows, cols = X.shape
    Xt = X.T
    out = pl.pallas_call(
        _colmax_body,
        in_specs=[pl.BlockSpec(memory_space=pl.ANY)],
        out_specs=pl.BlockSpec(memory_space=pltpu.VMEM),
        out_shape=jax.ShapeDtypeStruct((1, rows), X.dtype),
        scratch_shapes=[
            pltpu.VMEM((32, rows), X.dtype),
            pltpu.VMEM((_NBUF, _CHUNK, rows), X.dtype),
            pltpu.SemaphoreType.DMA((_NBUF,)),
        ],
    )(Xt)
    return out.reshape(rows)


# speedup vs baseline: 1.0755x; 1.0755x over previous
import jax
import jax.numpy as jnp
from jax.experimental import pallas as pl
from jax.experimental.pallas import tpu as pltpu

_CHUNK = 4000
_NBUF = 12


def _colmax_body(x_hbm, o_ref, acc, buf, sem):
    n = x_hbm.shape[0] // _CHUNK

    def copy(i):
        return pltpu.make_async_copy(
            x_hbm.at[pl.ds(i * _CHUNK, _CHUNK), :],
            buf.at[i % _NBUF],
            sem.at[i % _NBUF],
        )

    for i in range(min(_NBUF, n)):
        copy(i).start()
    acc[...] = jnp.full(acc.shape, -jnp.inf, acc.dtype)
    for i in range(n):
        copy(i).wait()
        chunk = buf[i % _NBUF].reshape(_CHUNK // 32, 32, 128)
        acc[...] = jnp.maximum(acc[...], jnp.max(chunk, axis=0))
        j = i + _NBUF
        if j < n:
            copy(j).start()
    a = jnp.max(acc[...].reshape(4, 8, 128), axis=0)
    o_ref[...] = jnp.max(a, axis=0, keepdims=True)


def kernel(X):
    rows, cols = X.shape
    Xt = X.T
    out = pl.pallas_call(
        _colmax_body,
        in_specs=[pl.BlockSpec(memory_space=pl.ANY)],
        out_specs=pl.BlockSpec(memory_space=pltpu.VMEM),
        out_shape=jax.ShapeDtypeStruct((1, rows), X.dtype),
        scratch_shapes=[
            pltpu.VMEM((32, rows), X.dtype),
            pltpu.VMEM((_NBUF, _CHUNK, rows), X.dtype),
            pltpu.SemaphoreType.DMA((_NBUF,)),
        ],
    )(Xt)
    return out.reshape(rows)


# speedup vs baseline: 1.1016x; 1.0243x over previous
import jax
import jax.numpy as jnp
from jax.experimental import pallas as pl
from jax.experimental.pallas import tpu as pltpu

_CHUNK = 800
_NBUF = 20


def _colmax_body(x_hbm, o_ref, acc, buf, sem):
    n = x_hbm.shape[0] // _CHUNK

    def copy(i):
        return pltpu.make_async_copy(
            x_hbm.at[pl.ds(i * _CHUNK, _CHUNK), :],
            buf.at[i % _NBUF],
            sem.at[i % _NBUF],
        )

    for i in range(min(_NBUF, n)):
        copy(i).start()
    acc[...] = jnp.full(acc.shape, -jnp.inf, acc.dtype)
    for i in range(n):
        copy(i).wait()
        chunk = buf[i % _NBUF].reshape(_CHUNK // 32, 32, 128)
        acc[...] = jnp.maximum(acc[...], jnp.max(chunk, axis=0))
        j = i + _NBUF
        if j < n:
            copy(j).start()
    a = jnp.max(acc[...].reshape(4, 8, 128), axis=0)
    o_ref[...] = jnp.max(a, axis=0, keepdims=True)


def kernel(X):
    rows, cols = X.shape
    Xt = X.T
    out = pl.pallas_call(
        _colmax_body,
        in_specs=[pl.BlockSpec(memory_space=pl.ANY)],
        out_specs=pl.BlockSpec(memory_space=pltpu.VMEM),
        out_shape=jax.ShapeDtypeStruct((1, rows), X.dtype),
        scratch_shapes=[
            pltpu.VMEM((32, rows), X.dtype),
            pltpu.VMEM((_NBUF, _CHUNK, rows), X.dtype),
            pltpu.SemaphoreType.DMA((_NBUF,)),
        ],
    )(Xt)
    return out.reshape(rows)
